# baseline (speedup 1.0000x reference)
import functools

import jax
import jax.numpy as jnp
from jax import lax
from jax.experimental import pallas as pl
from jax.experimental.pallas import tpu as pltpu
from jax.experimental.pallas import tpu_sc as plsc

L = 16
NUM_TYPES = 64
TC_BLOCK = 8192


def _sc_scale_shift(x, sc, sh, sp, boff, a, chunk, nc, ns):
    iters = chunk // L

    mesh = plsc.VectorSubcoreMesh(core_axis_name="c", subcore_axis_name="s")

    @functools.partial(
        pl.kernel,
        mesh=mesh,
        out_type=jax.ShapeDtypeStruct((a,), jnp.float32),
        compiler_params=pltpu.CompilerParams(needs_layout_passes=False),
        scratch_types=[
            pltpu.VMEM((chunk,), jnp.int32),
            pltpu.VMEM((chunk,), jnp.float32),
            pltpu.VMEM((chunk,), jnp.float32),
            pltpu.VMEM((NUM_TYPES,), jnp.float32),
            pltpu.VMEM((NUM_TYPES,), jnp.float32),
            pltpu.SemaphoreType.DMA,
        ],
    )
    def run(x_hbm, sc_hbm, sh_hbm, sp_hbm, out_hbm, sp_v, x_v, o_v, sc_v, sh_v, sem):
        wid = lax.axis_index("s") * nc + lax.axis_index("c")
        rel = jnp.minimum(wid * chunk, a - chunk)
        base = boff + rel

        cps = [
            pltpu.async_copy(sp_hbm.at[pl.ds(base, chunk)], sp_v, sem),
            pltpu.async_copy(x_hbm.at[pl.ds(base, chunk)], x_v, sem),
            pltpu.async_copy(sc_hbm, sc_v, sem),
            pltpu.async_copy(sh_hbm, sh_v, sem),
        ]
        for cp in cps:
            cp.wait()

        def body(i, carry):
            sl = pl.ds(i * L, L)
            idx = sp_v[sl]
            s = plsc.load_gather(sc_v, [idx])
            t = plsc.load_gather(sh_v, [idx])
            o_v[sl] = s * x_v[sl] + t
            return carry

        lax.fori_loop(0, iters, body, 0)

        pltpu.sync_copy(o_v, out_hbm.at[pl.ds(rel, chunk)])

    return run(x, sc, sh, sp)


def _tc_body(sc_ref, sh_ref, x_ref, sp_ref, o_ref):
    x = x_ref[...]
    idx = sp_ref[...]
    s = jnp.full(x.shape, sc_ref[0], dtype=jnp.float32)
    h = jnp.full(x.shape, sh_ref[0], dtype=jnp.float32)
    for t in range(1, NUM_TYPES):
        m = idx == t
        s = jnp.where(m, sc_ref[t], s)
        h = jnp.where(m, sh_ref[t], h)
    o_ref[...] = s * x + h


def _tc_scale_shift(x, sc, sh, sp, b):
    grid = (b // TC_BLOCK,)
    return pl.pallas_call(
        _tc_body,
        grid=grid,
        in_specs=[
            pl.BlockSpec(memory_space=pltpu.SMEM),
            pl.BlockSpec(memory_space=pltpu.SMEM),
            pl.BlockSpec((TC_BLOCK,), lambda i: (i,)),
            pl.BlockSpec((TC_BLOCK,), lambda i: (i,)),
        ],
        out_specs=pl.BlockSpec((TC_BLOCK,), lambda i: (i,)),
        out_shape=jax.ShapeDtypeStruct((b,), jnp.float32),
    )(sc, sh, x, sp)


def kernel(in_field, scales, shifts, species_idx, ptr):
    del ptr
    n = in_field.shape[0]
    x = in_field.reshape(-1)
    sp = species_idx.reshape(-1).astype(jnp.int32)
    sc = scales.reshape(-1).astype(jnp.float32)
    sh = shifts.reshape(-1).astype(jnp.float32)

    info = plsc.get_sparse_core_info()
    nc, ns = info.num_cores, info.num_subcores
    nw = nc * ns

    assert n % L == 0
    b = max(TC_BLOCK, (n * 2 // 3) // TC_BLOCK * TC_BLOCK)
    a = n - b

    per_worker = -(-a // nw)
    chunk = -(-per_worker // L) * L
    chunk = min(chunk, a)

    out_tc = _tc_scale_shift(x, sc, sh, sp, b)
    out_sc = _sc_scale_shift(x, sc, sh, sp, b, a, chunk, nc, ns)
    out = jnp.concatenate([out_tc, out_sc])
    return out.reshape(n, 1)


# speedup vs baseline: 1.0426x; 1.0426x over previous
import functools

import jax
import jax.numpy as jnp
from jax import lax
from jax.experimental import pallas as pl
from jax.experimental.pallas import tpu as pltpu
from jax.experimental.pallas import tpu_sc as plsc

L = 16


def _scale_shift_call(x, sc, sh, sp, n, chunk, nc, ns):
    half = chunk // 2
    iters = half // L

    mesh = plsc.VectorSubcoreMesh(core_axis_name="c", subcore_axis_name="s")

    @functools.partial(
        pl.kernel,
        mesh=mesh,
        out_type=jax.ShapeDtypeStruct((n,), jnp.float32),
        compiler_params=pltpu.CompilerParams(needs_layout_passes=False),
        scratch_types=[
            pltpu.VMEM((chunk,), jnp.int32),
            pltpu.VMEM((chunk,), jnp.float32),
            pltpu.VMEM((chunk,), jnp.float32),
            pltpu.VMEM((64,), jnp.float32),
            pltpu.VMEM((64,), jnp.float32),
            pltpu.SemaphoreType.DMA,
            pltpu.SemaphoreType.DMA,
            pltpu.SemaphoreType.DMA,
        ],
    )
    def run(
        x_hbm, sc_hbm, sh_hbm, sp_hbm, out_hbm,
        sp_v, x_v, o_v, sc_v, sh_v, sem0, sem1, osem,
    ):
        wid = lax.axis_index("s") * nc + lax.axis_index("c")
        base = jnp.minimum(wid * chunk, n - chunk)

        h0 = [
            pltpu.async_copy(sp_hbm.at[pl.ds(base, half)], sp_v.at[pl.ds(0, half)], sem0),
            pltpu.async_copy(x_hbm.at[pl.ds(base, half)], x_v.at[pl.ds(0, half)], sem0),
            pltpu.async_copy(sc_hbm, sc_v, sem0),
            pltpu.async_copy(sh_hbm, sh_v, sem0),
        ]
        h1 = [
            pltpu.async_copy(
                sp_hbm.at[pl.ds(base + half, half)], sp_v.at[pl.ds(half, half)], sem1
            ),
            pltpu.async_copy(
                x_hbm.at[pl.ds(base + half, half)], x_v.at[pl.ds(half, half)], sem1
            ),
        ]

        def body(start):
            def it(i, carry):
                sl = pl.ds(start + i * L, L)
                idx = sp_v[sl]
                s = plsc.load_gather(sc_v, [idx])
                t = plsc.load_gather(sh_v, [idx])
                o_v[sl] = s * x_v[sl] + t
                return carry

            lax.fori_loop(0, iters, it, 0)

        for cp in h0:
            cp.wait()
        body(0)
        out0 = pltpu.async_copy(
            o_v.at[pl.ds(0, half)], out_hbm.at[pl.ds(base, half)], osem
        )
        for cp in h1:
            cp.wait()
        body(half)
        out1 = pltpu.async_copy(
            o_v.at[pl.ds(half, half)], out_hbm.at[pl.ds(base + half, half)], osem
        )
        out0.wait()
        out1.wait()

    return run(x, sc, sh, sp)


def kernel(in_field, scales, shifts, species_idx, ptr):
    del ptr
    n0 = in_field.shape[0]
    x = in_field.reshape(-1)
    sp = species_idx.reshape(-1).astype(jnp.int32)
    sc = scales.reshape(-1).astype(jnp.float32)
    sh = shifts.reshape(-1).astype(jnp.float32)

    info = plsc.get_sparse_core_info()
    nc, ns = info.num_cores, info.num_subcores
    nw = nc * ns

    n = n0
    if n % L != 0:
        n = (n0 + L - 1) // L * L
        x = jnp.pad(x, (0, n - n0))
        sp = jnp.pad(sp, (0, n - n0))

    per_worker = -(-n // nw)
    chunk = -(-per_worker // (2 * L)) * (2 * L)
    chunk = min(chunk, n)

    out = _scale_shift_call(x, sc, sh, sp, n, chunk, nc, ns)
    return out[:n0].reshape(n0, 1)


# speedup vs baseline: 1.0643x; 1.0208x over previous
import functools

import jax
import jax.numpy as jnp
from jax import lax
from jax.experimental import pallas as pl
from jax.experimental.pallas import tpu as pltpu
from jax.experimental.pallas import tpu_sc as plsc

L = 16
NUM_TYPES = 64


def _scale_shift_call(x, sc, sh, sp, n, chunk, nc, ns):
    iters = chunk // L

    mesh = plsc.VectorSubcoreMesh(core_axis_name="c", subcore_axis_name="s")

    @functools.partial(
        pl.kernel,
        mesh=mesh,
        out_type=jax.ShapeDtypeStruct((n,), jnp.float32),
        compiler_params=pltpu.CompilerParams(
            needs_layout_passes=False, use_tc_tiling_on_sc=False
        ),
        scratch_types=[
            pltpu.VMEM((chunk,), jnp.int32),
            pltpu.VMEM((chunk,), jnp.float32),
            pltpu.VMEM((chunk,), jnp.float32),
            pltpu.VMEM((NUM_TYPES,), jnp.float32),
            pltpu.VMEM((NUM_TYPES,), jnp.float32),
            pltpu.SemaphoreType.DMA,
        ],
    )
    def run(x_hbm, sc_hbm, sh_hbm, sp_hbm, out_hbm, sp_v, x_v, o_v, sc_v, sh_v, sem):
        wid = lax.axis_index("s") * nc + lax.axis_index("c")
        base = jnp.minimum(wid * chunk, n - chunk)

        cps = [
            pltpu.async_copy(sp_hbm.at[pl.ds(base, chunk)], sp_v, sem),
            pltpu.async_copy(x_hbm.at[pl.ds(base, chunk)], x_v, sem),
            pltpu.async_copy(sc_hbm, sc_v, sem),
            pltpu.async_copy(sh_hbm, sh_v, sem),
        ]
        for cp in cps:
            cp.wait()

        def body(i, carry):
            sl = pl.ds(i * L, L)
            idx = sp_v[sl]
            s = plsc.load_gather(sc_v, [idx])
            t = plsc.load_gather(sh_v, [idx])
            o_v[sl] = s * x_v[sl] + t
            return carry

        lax.fori_loop(0, iters, body, 0)

        pltpu.sync_copy(o_v, out_hbm.at[pl.ds(base, chunk)])

    return run(x, sc, sh, sp)


def kernel(in_field, scales, shifts, species_idx, ptr):
    del ptr
    n = in_field.shape[0]
    x = in_field.reshape(-1)
    sp = species_idx.reshape(-1).astype(jnp.int32)
    sc = scales.reshape(-1).astype(jnp.float32)
    sh = shifts.reshape(-1).astype(jnp.float32)

    info = plsc.get_sparse_core_info()
    nc, ns = info.num_cores, info.num_subcores
    nw = nc * ns

    assert n % L == 0
    per_worker = -(-n // nw)
    chunk = -(-per_worker // L) * L
    chunk = min(chunk, n)

    out = _scale_shift_call(x, sc, sh, sp, n, chunk, nc, ns)
    return out.reshape(n, 1)


# speedup vs baseline: 1.1149x; 1.0475x over previous
import functools

import jax
import jax.numpy as jnp
from jax import lax
from jax.experimental import pallas as pl
from jax.experimental.pallas import tpu as pltpu
from jax.experimental.pallas import tpu_sc as plsc

L = 16
NUM_TYPES = 64


def _scale_shift_call(x, sc, sh, sp, n, chunk, nc, ns):
    iters = chunk // L

    mesh = plsc.VectorSubcoreMesh(core_axis_name="c", subcore_axis_name="s")

    @functools.partial(
        pl.kernel,
        mesh=mesh,
        out_type=jax.ShapeDtypeStruct((1, n), jnp.float32),
        compiler_params=pltpu.CompilerParams(
            needs_layout_passes=False, use_tc_tiling_on_sc=False
        ),
        scratch_types=[
            pltpu.VMEM((chunk,), jnp.int32),
            pltpu.VMEM((chunk,), jnp.float32),
            pltpu.VMEM((chunk,), jnp.float32),
            pltpu.VMEM((NUM_TYPES,), jnp.float32),
            pltpu.VMEM((NUM_TYPES,), jnp.float32),
            pltpu.SemaphoreType.DMA,
        ],
    )
    def run(x_hbm, sc_hbm, sh_hbm, sp_hbm, out_hbm, sp_v, x_v, o_v, sc_v, sh_v, sem):
        wid = lax.axis_index("s") * nc + lax.axis_index("c")
        base = jnp.minimum(wid * chunk, n - chunk)

        cps = [
            pltpu.async_copy(sp_hbm.at[0, pl.ds(base, chunk)], sp_v, sem),
            pltpu.async_copy(x_hbm.at[0, pl.ds(base, chunk)], x_v, sem),
            pltpu.async_copy(sc_hbm.at[0, :], sc_v, sem),
            pltpu.async_copy(sh_hbm.at[0, :], sh_v, sem),
        ]
        for cp in cps:
            cp.wait()

        def body(i, carry):
            sl = pl.ds(i * L, L)
            idx = sp_v[sl]
            s = plsc.load_gather(sc_v, [idx])
            t = plsc.load_gather(sh_v, [idx])
            o_v[sl] = s * x_v[sl] + t
            return carry

        lax.fori_loop(0, iters, body, 0)

        pltpu.sync_copy(o_v, out_hbm.at[0, pl.ds(base, chunk)])

    return run(x, sc, sh, sp)


def kernel(in_field, scales, shifts, species_idx, ptr):
    del ptr
    n = in_field.shape[0]
    x = in_field.reshape(1, n)
    sp = species_idx.reshape(1, n).astype(jnp.int32)

    info = plsc.get_sparse_core_info()
    nc, ns = info.num_cores, info.num_subcores
    nw = nc * ns

    assert n % L == 0
    per_worker = -(-n // nw)
    chunk = -(-per_worker // L) * L
    chunk = min(chunk, n)

    out = _scale_shift_call(x, scales, shifts, sp, n, chunk, nc, ns)
    return out.reshape(n, 1)
